# baseline (speedup 1.0000x reference)
import functools

import jax
import jax.numpy as jnp
from jax import lax
from jax.experimental import pallas as pl
from jax.experimental.pallas import tpu as pltpu
from jax.experimental.pallas import tpu_sc as plsc

_DIMS = 4096
_T = 64
_K = 8
_ROWS = 1024
_N = 8192
_CHUNKS = 2
_NC = _N // _CHUNKS
_NW = 32
_TPW = _NC // _NW
_LOW6 = ~63
_FLIP = 0x7FFFFFFF


def _tc_logits_body(x_ref, w_ref, b_ref, o_ref):
    z = lax.dot_general(
        x_ref[...], w_ref[...],
        dimension_numbers=(((1,), (1,)), ((), ())),
        preferred_element_type=jnp.float32,
    ) + b_ref[...]
    o_ref[...] = z.T


def _key(cvec, t):
    bits = lax.bitcast_convert_type(cvec, jnp.int32)
    skey = jnp.where(bits >= 0, bits, bits ^ _FLIP)
    return (skey & _LOW6) | (_T - 1 - t)


def _sc_route_body(zt_hbm, out_hbm, slab, obuf):
    wid = lax.axis_index("s") * 2 + lax.axis_index("c")
    base = wid * _TPW
    pltpu.sync_copy(zt_hbm.at[:, pl.ds(base, _TPW)], slab)

    def group(gi, carry):
        col = gi * 16
        imin = jnp.full((16,), jnp.iinfo(jnp.int32).min, jnp.int32)
        ms = [imin] * _K
        for t in range(_T):
            v = _key(slab[t, pl.ds(col, 16)], t)
            for i in range(_K):
                hi = jnp.maximum(ms[i], v)
                v = jnp.minimum(ms[i], v)
                ms[i] = hi
        thr = ms[_K - 1]
        sk0 = ms[0] & _LOW6
        m_f = lax.bitcast_convert_type(
            jnp.where(sk0 >= 0, sk0, sk0 ^ _FLIP), jnp.float32)
        den = jnp.zeros((16,), jnp.float32)
        num = jnp.zeros((16,), jnp.float32)
        for t in range(_T):
            cvec = slab[t, pl.ds(col, 16)]
            gvec = slab[_T + t, pl.ds(col, 16)]
            sel = _key(cvec, t) >= thr
            e = jnp.where(sel, jnp.exp(cvec - m_f), 0.0)
            den = den + e
            sg = 1.0 / (1.0 + jnp.exp(-gvec))
            num = num + e * sg
        obuf[pl.ds(col, 16)] = num / den
        return carry

    lax.fori_loop(0, _TPW // 16, group, 0)
    pltpu.sync_copy(obuf, out_hbm.at[pl.ds(base, _TPW)])


def _make_sc_route():
    return functools.partial(
        pl.kernel,
        mesh=plsc.VectorSubcoreMesh(core_axis_name="c", subcore_axis_name="s"),
        out_type=jax.ShapeDtypeStruct((_NC,), jnp.float32),
        scratch_types=[
            pltpu.VMEM((2 * _T, _TPW), jnp.float32),
            pltpu.VMEM((_TPW,), jnp.float32),
        ],
    )(_sc_route_body)


def kernel(x, Wc, bc, Wg, bg):
    B, S, D = x.shape
    n = B * S
    xf = x.reshape(n, D)
    w = jnp.concatenate([Wc, Wg], axis=0)
    b = jnp.concatenate([bc, bg], axis=0)[None, :]
    route = _make_sc_route()
    steps = _NC // _ROWS
    zts = []
    for ci in range(_CHUNKS):
        zts.append(pl.pallas_call(
            _tc_logits_body,
            grid=(steps,),
            in_specs=[
                pl.BlockSpec((_ROWS, D), lambda i, ci=ci: (ci * steps + i, 0)),
                pl.BlockSpec((2 * _T, D), lambda i: (0, 0)),
                pl.BlockSpec((1, 2 * _T), lambda i: (0, 0)),
            ],
            out_specs=pl.BlockSpec((2 * _T, _ROWS), lambda i: (0, i)),
            out_shape=jax.ShapeDtypeStruct((2 * _T, _NC), jnp.float32),
        )(xf, w, b))
    outs = [route(zt) for zt in zts]
    out = jnp.concatenate(outs)
    return out.reshape(B, S, 1)


# speedup vs baseline: 1.0266x; 1.0266x over previous
import functools

import jax
import jax.numpy as jnp
from jax import lax
from jax.experimental import pallas as pl
from jax.experimental.pallas import tpu as pltpu
from jax.experimental.pallas import tpu_sc as plsc

_DIMS = 4096
_T = 64
_K = 8
_ROWS = 1024
_N = 8192
_CHUNKS = 2
_NC = _N // _CHUNKS
_NW = 32
_TPW = _NC // _NW
_LOW6 = ~63
_FLIP = 0x7FFFFFFF


def _tc_terms_body(x_ref, w_ref, b_ref, o_ref):
    z = lax.dot_general(
        x_ref[...], w_ref[...],
        dimension_numbers=(((1,), (1,)), ((), ())),
        preferred_element_type=jnp.float32,
    ) + b_ref[...]
    zt = z.T
    c = zt[:_T, :]
    sig = 1.0 / (1.0 + jnp.exp(-zt[_T:, :]))
    m = jnp.max(c, axis=0, keepdims=True)
    e = jnp.exp(c - m)
    p = e * sig
    bits = lax.bitcast_convert_type(c, jnp.int32)
    skey = jnp.where(bits >= 0, bits, bits ^ _FLIP)
    iota = lax.broadcasted_iota(jnp.int32, (_T, c.shape[1]), 0)
    key = (skey & _LOW6) | (_T - 1 - iota)
    kf = lax.bitcast_convert_type(key, jnp.float32)
    o_ref[...] = jnp.concatenate([e, p, kf], axis=0)


def _sc_route_body(terms_hbm, out_hbm, slab, obuf):
    wid = lax.axis_index("s") * 2 + lax.axis_index("c")
    base = wid * _TPW
    pltpu.sync_copy(terms_hbm.at[:, pl.ds(base, _TPW)], slab)

    def group(gi, carry):
        col = gi * 16
        imin = jnp.full((16,), jnp.iinfo(jnp.int32).min, jnp.int32)
        ms = [imin] * _K
        for t in range(_T):
            v = lax.bitcast_convert_type(
                slab[2 * _T + t, pl.ds(col, 16)], jnp.int32)
            for i in range(_K):
                hi = jnp.maximum(ms[i], v)
                v = jnp.minimum(ms[i], v)
                ms[i] = hi
        thr = ms[_K - 1]
        den = jnp.zeros((16,), jnp.float32)
        num = jnp.zeros((16,), jnp.float32)
        for t in range(_T):
            k = lax.bitcast_convert_type(
                slab[2 * _T + t, pl.ds(col, 16)], jnp.int32)
            sel = k >= thr
            den = den + jnp.where(sel, slab[t, pl.ds(col, 16)], 0.0)
            num = num + jnp.where(sel, slab[_T + t, pl.ds(col, 16)], 0.0)
        obuf[pl.ds(col, 16)] = num / den
        return carry

    lax.fori_loop(0, _TPW // 16, group, 0)
    pltpu.sync_copy(obuf, out_hbm.at[pl.ds(base, _TPW)])


def _make_sc_route():
    return functools.partial(
        pl.kernel,
        mesh=plsc.VectorSubcoreMesh(core_axis_name="c", subcore_axis_name="s"),
        out_type=jax.ShapeDtypeStruct((_NC,), jnp.float32),
        scratch_types=[
            pltpu.VMEM((3 * _T, _TPW), jnp.float32),
            pltpu.VMEM((_TPW,), jnp.float32),
        ],
    )(_sc_route_body)


def kernel(x, Wc, bc, Wg, bg):
    B, S, D = x.shape
    n = B * S
    xf = x.reshape(n, D)
    w = jnp.concatenate([Wc, Wg], axis=0)
    b = jnp.concatenate([bc, bg], axis=0)[None, :]
    route = _make_sc_route()
    steps = _NC // _ROWS
    zts = []
    for ci in range(_CHUNKS):
        zts.append(pl.pallas_call(
            _tc_terms_body,
            grid=(steps,),
            in_specs=[
                pl.BlockSpec((_ROWS, D), lambda i, ci=ci: (ci * steps + i, 0)),
                pl.BlockSpec((2 * _T, D), lambda i: (0, 0)),
                pl.BlockSpec((1, 2 * _T), lambda i: (0, 0)),
            ],
            out_specs=pl.BlockSpec((3 * _T, _ROWS), lambda i: (0, i)),
            out_shape=jax.ShapeDtypeStruct((3 * _T, _NC), jnp.float32),
        )(xf, w, b))
    outs = [route(zt) for zt in zts]
    out = jnp.concatenate(outs)
    return out.reshape(B, S, 1)


# speedup vs baseline: 1.0756x; 1.0477x over previous
import functools

import jax
import jax.numpy as jnp
from jax import lax
from jax.experimental import pallas as pl
from jax.experimental.pallas import tpu as pltpu
from jax.experimental.pallas import tpu_sc as plsc

_DIMS = 4096
_T = 64
_K = 8
_ROWS = 1024
_N = 8192
_CHUNKS = 1
_NC = _N // _CHUNKS
_NW = 32
_TPW = _NC // _NW
_LOW6 = ~63
_FLIP = 0x7FFFFFFF


def _tc_terms_body(x_ref, w_ref, b_ref, o_ref):
    z = lax.dot_general(
        x_ref[...], w_ref[...],
        dimension_numbers=(((1,), (1,)), ((), ())),
        preferred_element_type=jnp.float32,
    ) + b_ref[...]
    zt = z.T
    c = zt[:_T, :]
    sig = 1.0 / (1.0 + jnp.exp(-zt[_T:, :]))
    m = jnp.max(c, axis=0, keepdims=True)
    e = jnp.exp(c - m)
    p = e * sig
    bits = lax.bitcast_convert_type(c, jnp.int32)
    skey = jnp.where(bits >= 0, bits, bits ^ _FLIP)
    iota = lax.broadcasted_iota(jnp.int32, (_T, c.shape[1]), 0)
    key = (skey & _LOW6) | (_T - 1 - iota)
    kf = lax.bitcast_convert_type(key, jnp.float32)
    o_ref[...] = jnp.concatenate([e, p, kf], axis=0)


def _sc_route_body(terms_hbm, out_hbm, slab, obuf):
    wid = lax.axis_index("s") * 2 + lax.axis_index("c")
    base = wid * _TPW
    pltpu.sync_copy(terms_hbm.at[:, pl.ds(base, _TPW)], slab)

    def group(gi, carry):
        col = gi * 16
        imin = jnp.full((16,), jnp.iinfo(jnp.int32).min, jnp.int32)
        ms = [imin] * _K
        for t in range(_T):
            v = lax.bitcast_convert_type(
                slab[2 * _T + t, pl.ds(col, 16)], jnp.int32)
            for i in range(_K):
                hi = jnp.maximum(ms[i], v)
                v = jnp.minimum(ms[i], v)
                ms[i] = hi
        thr = ms[_K - 1]
        den = jnp.zeros((16,), jnp.float32)
        num = jnp.zeros((16,), jnp.float32)
        for t in range(_T):
            k = lax.bitcast_convert_type(
                slab[2 * _T + t, pl.ds(col, 16)], jnp.int32)
            sel = k >= thr
            den = den + jnp.where(sel, slab[t, pl.ds(col, 16)], 0.0)
            num = num + jnp.where(sel, slab[_T + t, pl.ds(col, 16)], 0.0)
        obuf[pl.ds(col, 16)] = num / den
        return carry

    lax.fori_loop(0, _TPW // 16, group, 0)
    pltpu.sync_copy(obuf, out_hbm.at[pl.ds(base, _TPW)])


def _make_sc_route():
    return functools.partial(
        pl.kernel,
        mesh=plsc.VectorSubcoreMesh(core_axis_name="c", subcore_axis_name="s"),
        out_type=jax.ShapeDtypeStruct((_NC,), jnp.float32),
        scratch_types=[
            pltpu.VMEM((3 * _T, _TPW), jnp.float32),
            pltpu.VMEM((_TPW,), jnp.float32),
        ],
    )(_sc_route_body)


def kernel(x, Wc, bc, Wg, bg):
    B, S, D = x.shape
    n = B * S
    xf = x.reshape(n, D)
    w = jnp.concatenate([Wc, Wg], axis=0)
    b = jnp.concatenate([bc, bg], axis=0)[None, :]
    route = _make_sc_route()
    steps = _NC // _ROWS
    zts = []
    for ci in range(_CHUNKS):
        zts.append(pl.pallas_call(
            _tc_terms_body,
            grid=(steps,),
            in_specs=[
                pl.BlockSpec((_ROWS, D), lambda i, ci=ci: (ci * steps + i, 0)),
                pl.BlockSpec((2 * _T, D), lambda i: (0, 0)),
                pl.BlockSpec((1, 2 * _T), lambda i: (0, 0)),
            ],
            out_specs=pl.BlockSpec((3 * _T, _ROWS), lambda i: (0, i)),
            out_shape=jax.ShapeDtypeStruct((3 * _T, _NC), jnp.float32),
        )(xf, w, b))
    outs = [route(zt) for zt in zts]
    out = jnp.concatenate(outs)
    return out.reshape(B, S, 1)


# speedup vs baseline: 1.0787x; 1.0029x over previous
import functools

import jax
import jax.numpy as jnp
from jax import lax
from jax.experimental import pallas as pl
from jax.experimental.pallas import tpu as pltpu
from jax.experimental.pallas import tpu_sc as plsc

_DIMS = 4096
_T = 64
_K = 8
_ROWS = 1024
_N = 8192
_CHUNKS = 1
_NC = _N // _CHUNKS
_NW = 32
_TPW = _NC // _NW
_LOW6 = ~63
_FLIP = 0x7FFFFFFF


def _tc_terms_body(x_ref, w_ref, b_ref, o_ref):
    z = lax.dot_general(
        x_ref[...], w_ref[...],
        dimension_numbers=(((1,), (1,)), ((), ())),
        preferred_element_type=jnp.float32,
    ) + b_ref[...]
    zt = z.T
    c = zt[:_T, :]
    sig = 1.0 / (1.0 + jnp.exp(-zt[_T:, :]))
    m = jnp.max(c, axis=0, keepdims=True)
    e = jnp.exp(c - m)
    p = e * sig
    bits = lax.bitcast_convert_type(c, jnp.int32)
    skey = jnp.where(bits >= 0, bits, bits ^ _FLIP)
    iota = lax.broadcasted_iota(jnp.int32, (_T, c.shape[1]), 0)
    key = (skey & _LOW6) | (_T - 1 - iota)
    kf = lax.bitcast_convert_type(key, jnp.float32)
    terms = jnp.concatenate([e, p, kf], axis=0)
    o_ref[...] = terms.reshape(3 * _T, -1, _TPW).transpose(1, 0, 2)


def _sc_route_body(terms_hbm, out_hbm, slab, obuf):
    wid = lax.axis_index("s") * 2 + lax.axis_index("c")
    base = wid * _TPW
    pltpu.sync_copy(terms_hbm.at[wid], slab)

    def group(gi, carry):
        col = gi * 16
        imin = jnp.full((16,), jnp.iinfo(jnp.int32).min, jnp.int32)
        ms = [imin] * _K
        for t in range(_T):
            v = lax.bitcast_convert_type(
                slab[2 * _T + t, pl.ds(col, 16)], jnp.int32)
            for i in range(_K):
                hi = jnp.maximum(ms[i], v)
                v = jnp.minimum(ms[i], v)
                ms[i] = hi
        thr = ms[_K - 1]
        den = jnp.zeros((16,), jnp.float32)
        num = jnp.zeros((16,), jnp.float32)
        for t in range(_T):
            k = lax.bitcast_convert_type(
                slab[2 * _T + t, pl.ds(col, 16)], jnp.int32)
            sel = k >= thr
            den = den + jnp.where(sel, slab[t, pl.ds(col, 16)], 0.0)
            num = num + jnp.where(sel, slab[_T + t, pl.ds(col, 16)], 0.0)
        obuf[pl.ds(col, 16)] = num / den
        return carry

    lax.fori_loop(0, _TPW // 16, group, 0)
    pltpu.sync_copy(obuf, out_hbm.at[pl.ds(base, _TPW)])


def _make_sc_route():
    return functools.partial(
        pl.kernel,
        mesh=plsc.VectorSubcoreMesh(core_axis_name="c", subcore_axis_name="s"),
        out_type=jax.ShapeDtypeStruct((_NC,), jnp.float32),
        scratch_types=[
            pltpu.VMEM((3 * _T, _TPW), jnp.float32),
            pltpu.VMEM((_TPW,), jnp.float32),
        ],
    )(_sc_route_body)


def kernel(x, Wc, bc, Wg, bg):
    B, S, D = x.shape
    n = B * S
    xf = x.reshape(n, D)
    w = jnp.concatenate([Wc, Wg], axis=0)
    b = jnp.concatenate([bc, bg], axis=0)[None, :]
    route = _make_sc_route()
    steps = _NC // _ROWS
    zts = []
    for ci in range(_CHUNKS):
        zts.append(pl.pallas_call(
            _tc_terms_body,
            grid=(steps,),
            in_specs=[
                pl.BlockSpec((_ROWS, D), lambda i, ci=ci: (ci * steps + i, 0)),
                pl.BlockSpec((2 * _T, D), lambda i: (0, 0)),
                pl.BlockSpec((1, 2 * _T), lambda i: (0, 0)),
            ],
            out_specs=pl.BlockSpec(
                (_ROWS // _TPW, 3 * _T, _TPW), lambda i: (i, 0, 0)),
            out_shape=jax.ShapeDtypeStruct(
                (_NC // _TPW, 3 * _T, _TPW), jnp.float32),
        )(xf, w, b))
    outs = [route(zt) for zt in zts]
    out = jnp.concatenate(outs)
    return out.reshape(B, S, 1)
